# Initial kernel scaffold; baseline (speedup 1.0000x reference)
import jax
import jax.numpy as jnp
from jax.experimental import pallas as pl


def kernel(input_x, table):
    raise NotImplementedError("write your pallas kernel here")


# speedup vs baseline: 1.8435x; 1.8435x over previous
import functools

import jax
import jax.numpy as jnp
from jax import lax
from jax.experimental import pallas as pl
from jax.experimental.pallas import tpu as pltpu
from jax.experimental.pallas import tpu_sc as plsc


@functools.lru_cache(maxsize=None)
def _make_gather(V, D, B):
    info = plsc.get_sparse_core_info()
    NC, NS = info.num_cores, info.num_subcores
    NW = NC * NS
    assert B % NW == 0
    b_per_w = B // NW
    C = 1024
    assert b_per_w % C == 0
    n_chunks = b_per_w // C
    mesh = plsc.VectorSubcoreMesh(core_axis_name="c", subcore_axis_name="s")

    @functools.partial(
        pl.kernel,
        mesh=mesh,
        out_type=jax.ShapeDtypeStruct((B, D), jnp.float32),
        compiler_params=pltpu.CompilerParams(use_tc_tiling_on_sc=False),
        scratch_types=[
            pltpu.VMEM((C,), jnp.int32),
            pltpu.VMEM((C, D), jnp.float32),
            pltpu.SemaphoreType.DMA,
        ],
    )
    def k(idx_hbm, table_hbm, out_hbm, idx_v, rows_v, sem):
        wid = lax.axis_index("s") * NC + lax.axis_index("c")
        w_base = wid * b_per_w

        def body(i, carry):
            base = w_base + i * C
            pltpu.sync_copy(idx_hbm.at[pl.ds(base, C)], idx_v)
            pltpu.async_copy(table_hbm.at[idx_v], rows_v, sem).wait()
            pltpu.sync_copy(rows_v, out_hbm.at[pl.ds(base, C)])
            return carry

        lax.fori_loop(0, n_chunks, body, 0)

    return k


def kernel(input_x, table):
    Bt, H = input_x.shape
    V, D = table.shape
    idx = input_x.reshape(-1).astype(jnp.int32)
    out = _make_gather(V, D, idx.shape[0])(idx, table)
    return out.reshape(Bt, H, D)


# speedup vs baseline: 1.8620x; 1.0101x over previous
import functools

import jax
import jax.numpy as jnp
from jax import lax
from jax.experimental import pallas as pl
from jax.experimental.pallas import tpu as pltpu
from jax.experimental.pallas import tpu_sc as plsc

_NBUF = 2


@functools.lru_cache(maxsize=None)
def _make_gather(V, D, B):
    info = plsc.get_sparse_core_info()
    NC, NS = info.num_cores, info.num_subcores
    NW = NC * NS
    assert B % NW == 0
    b_per_w = B // NW
    C = 800
    assert b_per_w % (C * _NBUF) == 0
    n_chunks = b_per_w // C
    n_groups = n_chunks // _NBUF
    mesh = plsc.VectorSubcoreMesh(core_axis_name="c", subcore_axis_name="s")

    @functools.partial(
        pl.kernel,
        mesh=mesh,
        out_type=jax.ShapeDtypeStruct((B, D), jnp.float32),
        compiler_params=pltpu.CompilerParams(use_tc_tiling_on_sc=False),
        scratch_types=[
            pltpu.VMEM((_NBUF, C), jnp.int32),
            pltpu.VMEM((_NBUF, C, D), jnp.float32),
            [pltpu.SemaphoreType.DMA] * _NBUF,
            [pltpu.SemaphoreType.DMA] * _NBUF,
        ],
    )
    def k(idx_hbm, table_hbm, out_hbm, idx_v, rows_v, gsem, ssem):
        wid = lax.axis_index("s") * NC + lax.axis_index("c")
        w_base = wid * b_per_w

        def start_gather(chunk, b):
            pltpu.sync_copy(idx_hbm.at[pl.ds(w_base + chunk * C, C)],
                            idx_v.at[b])
            pltpu.async_copy(table_hbm.at[idx_v.at[b]], rows_v.at[b], gsem[b])

        def wait_gather(b):
            pltpu.make_async_copy(table_hbm.at[idx_v.at[b]], rows_v.at[b],
                                  gsem[b]).wait()

        def start_store(chunk, b):
            pltpu.async_copy(rows_v.at[b],
                             out_hbm.at[pl.ds(w_base + chunk * C, C)], ssem[b])

        def wait_store(chunk, b):
            pltpu.make_async_copy(rows_v.at[b],
                                  out_hbm.at[pl.ds(w_base + chunk * C, C)],
                                  ssem[b]).wait()

        for b in range(_NBUF):
            start_gather(b, b)

        def body(g, carry):
            for b in range(_NBUF):
                i = g * _NBUF + b
                wait_gather(b)
                start_store(i, b)
                wait_store(i, b)
                start_gather(i + _NBUF, b)
            return carry

        lax.fori_loop(0, n_groups - 1, body, 0)

        for b in range(_NBUF):
            i = (n_groups - 1) * _NBUF + b
            wait_gather(b)
            pltpu.sync_copy(rows_v.at[b], out_hbm.at[pl.ds(w_base + i * C, C)])

    return k


def kernel(input_x, table):
    Bt, H = input_x.shape
    V, D = table.shape
    idx = input_x.reshape(-1).astype(jnp.int32)
    out = _make_gather(V, D, idx.shape[0])(idx, table)
    return out.reshape(Bt, H, D)
